# Initial kernel scaffold; baseline (speedup 1.0000x reference)
import jax
import jax.numpy as jnp
from jax.experimental import pallas as pl


def kernel(x, edge_index, W1, b1, W2, b2, W3, b3, Wd1, bd1, Wd2, bd2, Wd3, bd3):
    raise NotImplementedError("write your pallas kernel here")


# speedup vs baseline: 6.0581x; 6.0581x over previous
import functools

import jax
import jax.numpy as jnp
from jax import lax
from jax.experimental import pallas as pl
from jax.experimental.pallas import tpu as pltpu
from jax.experimental.pallas import tpu_sc as plsc

N = 10000
E = 320000
NC = 2
NS = 16
NW = NC * NS
K = 128
CPW = 80
E_PAD = NW * CPW * K
N_ACC = 10112
RPT = N_ACC // NS
DEG_W = 8
NBUF = 2


def _make_seg_pass(D, with_deg):
    out_type = [jax.ShapeDtypeStruct((NC, N_ACC, D), jnp.float32)]
    if with_deg:
        out_type.append(jax.ShapeDtypeStruct((NC, N_ACC, DEG_W), jnp.float32))
    scratch = [
        pltpu.VMEM((CPW, K), jnp.int32),
        pltpu.VMEM((CPW, K), jnp.int32),
        pltpu.VMEM((NBUF, K, D), jnp.float32),
        pltpu.VMEM_SHARED((N_ACC, D), jnp.float32),
        pltpu.SemaphoreType.DMA((NBUF,)),
    ]
    if with_deg:
        scratch.append(pltpu.VMEM((K, DEG_W), jnp.float32))
        scratch.append(pltpu.VMEM_SHARED((N_ACC, DEG_W), jnp.float32))

    def body(h_hbm, srcp_hbm, dstp_hbm, zeros_hbm, *rest):
        if with_deg:
            (zeros_deg_hbm, ones_hbm, out_hbm, deg_out_hbm,
             src_v, dst_v, bufs, acc, sems, ones_v, dacc) = rest
        else:
            (out_hbm, src_v, dst_v, bufs, acc, sems) = rest
        c = lax.axis_index("c")
        s = lax.axis_index("s")
        wid = s * NC + c

        pltpu.sync_copy(srcp_hbm.at[wid], src_v)
        pltpu.sync_copy(dstp_hbm.at[wid], dst_v)

        rows = pl.ds(s * RPT, RPT)
        pltpu.sync_copy(zeros_hbm.at[rows], acc.at[rows])
        if with_deg:
            pltpu.sync_copy(zeros_deg_hbm.at[rows], dacc.at[rows])
            pltpu.sync_copy(ones_hbm, ones_v)
        plsc.subcore_barrier()

        def gather_start(j, b):
            pltpu.make_async_copy(
                h_hbm.at[src_v.at[j]], bufs.at[b], sems.at[b]).start()

        for b in range(NBUF):
            gather_start(b, b)

        def step(i, _):
            for b in range(NBUF):
                j = i * NBUF + b
                pltpu.make_async_copy(
                    h_hbm.at[src_v.at[j]], bufs.at[b], sems.at[b]).wait()
                pltpu.sync_copy(bufs.at[b], acc.at[dst_v.at[j]], add=True)
                if with_deg:
                    pltpu.sync_copy(ones_v, dacc.at[dst_v.at[j]], add=True)

                @pl.when(j + NBUF < CPW)
                def _():
                    gather_start(j + NBUF, b)
            return _

        lax.fori_loop(0, CPW // NBUF, step, None)
        plsc.subcore_barrier()

        pltpu.sync_copy(acc.at[rows], out_hbm.at[c, rows])
        if with_deg:
            pltpu.sync_copy(dacc.at[rows], deg_out_hbm.at[c, rows])

    mesh = plsc.VectorSubcoreMesh(core_axis_name="c", subcore_axis_name="s",
                                  num_cores=NC, num_subcores=NS)
    return pl.kernel(body, out_type=tuple(out_type), mesh=mesh,
                     scratch_types=scratch,
                     compiler_params=pltpu.CompilerParams(
                         use_tc_tiling_on_sc=False))


_make_seg_pass = functools.lru_cache(maxsize=None)(_make_seg_pass)


def _row_blocks(nrows, width):
    return pl.BlockSpec((nrows, width), lambda i: (i, 0))


def _part_blocks(nrows, width):
    return pl.BlockSpec((NC, nrows, width), lambda i: (0, i, 0))


def _full(shape):
    return pl.BlockSpec(shape, lambda i: (0,) * len(shape))


_BLK = 1000
_GRID = N // _BLK


def _inv_deg(deg_ref):
    d = deg_ref[0, :, 0] + deg_ref[1, :, 0]
    return (1.0 / (d + 1.0))[:, None]


def _tc1_body(agga_ref, aggb_ref, deg_ref, x_ref, w1a_ref, w1b_ref, b1_ref,
              w2_ref, g2_ref):
    inv = _inv_deg(deg_ref)
    xb = x_ref[...]
    hna = (agga_ref[0] + agga_ref[1] + xb[:, :64]) * inv
    hnb = (aggb_ref[0] + aggb_ref[1] + xb[:, 64:]) * inv
    h1 = jnp.maximum(
        jnp.dot(hna, w1a_ref[...], preferred_element_type=jnp.float32)
        + jnp.dot(hnb, w1b_ref[...], preferred_element_type=jnp.float32)
        + b1_ref[...], 0.0)
    g2_ref[...] = jnp.dot(h1, w2_ref[...], preferred_element_type=jnp.float32)


def _tc2_body(q_ref, deg_ref, g2_ref, b2_ref, w3_ref, g3_ref):
    h2 = jnp.maximum(
        (q_ref[0] + q_ref[1] + g2_ref[...]) * _inv_deg(deg_ref)
        + b2_ref[...], 0.0)
    g3_ref[...] = jnp.dot(h2, w3_ref[...], preferred_element_type=jnp.float32)


def _tc3_body(r_ref, deg_ref, g3_ref, b3_ref, wd1_ref, bd1_ref, wd2_ref,
              bd2_ref, wd3_ref, bd3_ref, xen_ref, xde_ref):
    xen = ((r_ref[0] + r_ref[1] + g3_ref[...]) * _inv_deg(deg_ref)
           + b3_ref[...])
    xen_ref[...] = xen
    d = jnp.maximum(
        jnp.dot(xen, wd1_ref[...], preferred_element_type=jnp.float32)
        + bd1_ref[...], 0.0)
    d = jnp.maximum(
        jnp.dot(d, wd2_ref[...], preferred_element_type=jnp.float32)
        + bd2_ref[...], 0.0)
    xde_ref[...] = (jnp.dot(d, wd3_ref[...], preferred_element_type=jnp.float32)
                    + bd3_ref[...])


def kernel(x, edge_index, W1, b1, W2, b2, W3, b3, Wd1, bd1, Wd2, bd2, Wd3, bd3):
    src = edge_index[0]
    dst = edge_index[1]
    pad = E_PAD - E
    srcp = jnp.concatenate([src, jnp.zeros((pad,), jnp.int32)]).reshape(
        NW, CPW, K)
    dstp = jnp.concatenate([dst, jnp.full((pad,), N, jnp.int32)]).reshape(
        NW, CPW, K)
    z64 = jnp.zeros((N_ACC, 64), jnp.float32)
    z16 = jnp.zeros((N_ACC, 16), jnp.float32)
    zdeg = jnp.zeros((N_ACC, DEG_W), jnp.float32)
    ones = jnp.ones((K, DEG_W), jnp.float32)

    x0 = x[:, :64]
    x1 = x[:, 64:]
    agg1a, deg = _make_seg_pass(64, True)(x0, srcp, dstp, z64, zdeg, ones)
    (agg1b,) = _make_seg_pass(64, False)(x1, srcp, dstp, z64)

    g2 = pl.pallas_call(
        _tc1_body,
        grid=(_GRID,),
        in_specs=[_part_blocks(_BLK, 64), _part_blocks(_BLK, 64),
                  _part_blocks(_BLK, DEG_W),
                  _row_blocks(_BLK, 128), _full((64, 256)), _full((64, 256)),
                  _full((256,)), _full((256, 64))],
        out_specs=_row_blocks(_BLK, 64),
        out_shape=jax.ShapeDtypeStruct((N, 64), jnp.float32),
    )(agg1a, agg1b, deg, x, W1[:64], W1[64:], b1, W2)

    (agg2,) = _make_seg_pass(64, False)(g2, srcp, dstp, z64)

    g3 = pl.pallas_call(
        _tc2_body,
        grid=(_GRID,),
        in_specs=[_part_blocks(_BLK, 64), _part_blocks(_BLK, DEG_W),
                  _row_blocks(_BLK, 64), _full((64,)), _full((64, 16))],
        out_specs=_row_blocks(_BLK, 16),
        out_shape=jax.ShapeDtypeStruct((N, 16), jnp.float32),
    )(agg2, deg, g2, b2, W3)

    (agg3,) = _make_seg_pass(16, False)(g3, srcp, dstp, z16)

    x_en, x_de = pl.pallas_call(
        _tc3_body,
        grid=(_GRID,),
        in_specs=[_part_blocks(_BLK, 16), _part_blocks(_BLK, DEG_W),
                  _row_blocks(_BLK, 16), _full((16,)), _full((16, 64)),
                  _full((64,)), _full((64, 256)), _full((256,)),
                  _full((256, 128)), _full((128,))],
        out_specs=[_row_blocks(_BLK, 16), _row_blocks(_BLK, 128)],
        out_shape=[jax.ShapeDtypeStruct((N, 16), jnp.float32),
                   jax.ShapeDtypeStruct((N, 128), jnp.float32)],
    )(agg3, deg, g3, b3, Wd1, bd1, Wd2, bd2, Wd3, bd3)

    return (x_en, x_de)
